# baseline (speedup 1.0000x reference)
import functools

import jax
import jax.numpy as jnp
from jax.experimental import pallas as pl
from jax.experimental.pallas import tpu as pltpu

N_TOKENS = 4096
N_CODES = 16384
DIM = 2048
BM = 2048
BN = 256
NJ = N_CODES // BN


def _vq_argmin_body(x_ref, e_ref, x2_ref, idx_ref,
                    accv_ref, acci_ref, winm_ref, winb_ref,
                    dotp_ref, e2p_ref):
    j = pl.program_id(1)

    e_blk = e_ref[...]
    dot = jax.lax.dot_general(
        x_ref[...], e_blk, (((1,), (1,)), ((), ())),
        preferred_element_type=jnp.float32)
    p = jax.lax.rem(j, 2)
    dotp_ref[pl.ds(p, 1), :, :] = dot[None]
    e2p_ref[pl.ds(p, 1), :, :] = jnp.sum(e_blk * e_blk, axis=1)[None, None, :]

    @pl.when(j == 0)
    def _():
        accv_ref[...] = jnp.full_like(accv_ref[...], jnp.inf)
        acci_ref[...] = jnp.zeros_like(acci_ref[...])

    b = j - 1
    q = jax.lax.rem(j + 1, 2)
    d = ((x2_ref[...] + e2p_ref[pl.ds(q, 1), 0, :])
         - 2.0 * dotp_ref[pl.ds(q, 1), :, :][0])
    b_start = b % 5 == 0
    upd = jnp.logical_or(b_start, d < winm_ref[...])
    winm_ref[...] = jnp.where(upd, d, winm_ref[...])
    winb_ref[...] = jnp.where(upd, b, winb_ref[...])

    win_end = jnp.logical_and(
        b >= 0,
        jnp.logical_or(jnp.logical_and(b % 5 == 4, b < 60), b == NJ - 1))

    @pl.when(win_end)
    def _():
        winm = winm_ref[...]
        wmin = jnp.min(winm, axis=1, keepdims=True)
        cols = jax.lax.broadcasted_iota(jnp.int32, winm.shape, 1)
        g = winb_ref[...] * BN + cols
        widx = jnp.min(jnp.where(winm == wmin, g, N_CODES),
                       axis=1, keepdims=True)
        aupd = wmin < accv_ref[...]
        accv_ref[...] = jnp.where(
            aupd, wmin.astype(jnp.bfloat16).astype(jnp.float32),
            accv_ref[...])
        acci_ref[...] = jnp.where(aupd, widx, acci_ref[...])

    @pl.when(j == NJ)
    def _():
        idx_ref[...] = acci_ref[...]


@jax.jit
def kernel(x, embedding):
    x2 = jnp.sum(x ** 2, axis=1, keepdims=True)

    grid = (N_TOKENS // BM, NJ + 1)
    idx2d = pl.pallas_call(
        _vq_argmin_body,
        grid=grid,
        in_specs=[
            pl.BlockSpec((BM, DIM), lambda i, j: (i, 0)),
            pl.BlockSpec((BN, DIM), lambda i, j: (min(j, NJ - 1)
                                                  if isinstance(j, int)
                                                  else jnp.minimum(j, NJ - 1),
                                                  0)),
            pl.BlockSpec((BM, 1), lambda i, j: (i, 0)),
        ],
        out_specs=pl.BlockSpec((BM, 1), lambda i, j: (i, 0)),
        out_shape=jax.ShapeDtypeStruct((N_TOKENS, 1), jnp.int32),
        scratch_shapes=[
            pltpu.VMEM((BM, 1), jnp.float32),
            pltpu.VMEM((BM, 1), jnp.int32),
            pltpu.VMEM((BM, BN), jnp.float32),
            pltpu.VMEM((BM, BN), jnp.int32),
            pltpu.VMEM((2, BM, BN), jnp.float32),
            pltpu.VMEM((2, 1, BN), jnp.float32),
        ],
        compiler_params=pltpu.CompilerParams(
            dimension_semantics=("parallel", "arbitrary")),
    )(x, embedding, x2)

    indices = idx2d[:, 0]
    z_q = jnp.take(embedding, indices, axis=0)
    return (z_q, indices)


# speedup vs baseline: 1.0662x; 1.0662x over previous
import functools

import jax
import jax.numpy as jnp
from jax.experimental import pallas as pl
from jax.experimental.pallas import tpu as pltpu

N_TOKENS = 4096
N_CODES = 16384
DIM = 2048
BM = 2048
BN = 512
NJ = N_CODES // BN


def _piece(d, j, c0, width, start, close, accv_ref, acci_ref,
           winv_ref, wini_ref):
    dp = d[:, c0:c0 + width]
    pmin = jnp.min(dp, axis=1, keepdims=True)
    cols = jax.lax.broadcasted_iota(jnp.int32, dp.shape, 1)
    ploc = jnp.min(jnp.where(dp == pmin, cols, N_CODES),
                   axis=1, keepdims=True)
    pidx = ploc + (j * BN + c0)
    if start:
        winv_ref[...] = pmin
        wini_ref[...] = pidx
    else:
        upd = pmin < winv_ref[...]
        winv_ref[...] = jnp.where(upd, pmin, winv_ref[...])
        wini_ref[...] = jnp.where(upd, pidx, wini_ref[...])
    if close:
        aupd = winv_ref[...] < accv_ref[...]
        accv_ref[...] = jnp.where(
            aupd, winv_ref[...].astype(jnp.bfloat16).astype(jnp.float32),
            accv_ref[...])
        acci_ref[...] = jnp.where(aupd, wini_ref[...], acci_ref[...])


def _vq_argmin_body(x_ref, e_ref, x2_ref, idx_ref,
                    accv_ref, acci_ref, winv_ref, wini_ref):
    j = pl.program_id(1)
    e_blk = e_ref[...]
    dot = jax.lax.dot_general(
        x_ref[...], e_blk, (((1,), (1,)), ((), ())),
        preferred_element_type=jnp.float32)
    e2 = jnp.sum(e_blk * e_blk, axis=1)[None, :]
    d = (x2_ref[...] + e2) - 2.0 * dot

    @pl.when(j == 0)
    def _():
        accv_ref[...] = jnp.full_like(accv_ref[...], jnp.inf)
        acci_ref[...] = jnp.zeros_like(acci_ref[...])

    refs = (accv_ref, acci_ref, winv_ref, wini_ref)
    patterns = {
        0: [(0, 512, True, False)],
        1: [(0, 512, False, False)],
        2: [(0, 256, False, True), (256, 256, True, False)],
        3: [(0, 512, False, False)],
        4: [(0, 512, False, True)],
    }
    for pat, pieces in patterns.items():
        cond = j % 5 == pat
        if pat == 1:
            cond = jnp.logical_and(cond, j != NJ - 1)

        @pl.when(cond)
        def _(pieces=pieces):
            for (c0, w, st, cl) in pieces:
                _piece(d, j, c0, w, st, cl, *refs)

    @pl.when(j == NJ - 1)
    def _():
        _piece(d, j, 0, 512, False, True, *refs)
        idx_ref[...] = acci_ref[...]


@jax.jit
def kernel(x, embedding):
    x2 = jnp.sum(x ** 2, axis=1, keepdims=True)

    grid = (N_TOKENS // BM, NJ)
    idx2d = pl.pallas_call(
        _vq_argmin_body,
        grid=grid,
        in_specs=[
            pl.BlockSpec((BM, DIM), lambda i, j: (i, 0)),
            pl.BlockSpec((BN, DIM), lambda i, j: (j, 0)),
            pl.BlockSpec((BM, 1), lambda i, j: (i, 0)),
        ],
        out_specs=pl.BlockSpec((BM, 1), lambda i, j: (i, 0)),
        out_shape=jax.ShapeDtypeStruct((N_TOKENS, 1), jnp.int32),
        scratch_shapes=[
            pltpu.VMEM((BM, 1), jnp.float32),
            pltpu.VMEM((BM, 1), jnp.int32),
            pltpu.VMEM((BM, 1), jnp.float32),
            pltpu.VMEM((BM, 1), jnp.int32),
        ],
        compiler_params=pltpu.CompilerParams(
            dimension_semantics=("parallel", "arbitrary")),
    )(x, embedding, x2)

    indices = idx2d[:, 0]
    z_q = jnp.take(embedding, indices, axis=0)
    return (z_q, indices)


# speedup vs baseline: 1.1256x; 1.0557x over previous
import functools

import jax
import jax.numpy as jnp
from jax import lax
from jax.experimental import pallas as pl
from jax.experimental.pallas import tpu as pltpu
from jax.experimental.pallas import tpu_sc as plsc

N_TOKENS = 4096
N_CODES = 16384
DIM = 2048
BM = 2048
BN = 512
NJ = N_CODES // BN


def _piece(d, j, c0, width, start, close, accv_ref, acci_ref,
           winv_ref, wini_ref):
    dp = d[:, c0:c0 + width]
    pmin = jnp.min(dp, axis=1, keepdims=True)
    cols = jax.lax.broadcasted_iota(jnp.int32, dp.shape, 1)
    ploc = jnp.min(jnp.where(dp == pmin, cols, N_CODES),
                   axis=1, keepdims=True)
    pidx = ploc + (j * BN + c0)
    if start:
        winv_ref[...] = pmin
        wini_ref[...] = pidx
    else:
        upd = pmin < winv_ref[...]
        winv_ref[...] = jnp.where(upd, pmin, winv_ref[...])
        wini_ref[...] = jnp.where(upd, pidx, wini_ref[...])
    if close:
        aupd = winv_ref[...] < accv_ref[...]
        accv_ref[...] = jnp.where(
            aupd, winv_ref[...].astype(jnp.bfloat16).astype(jnp.float32),
            accv_ref[...])
        acci_ref[...] = jnp.where(aupd, wini_ref[...], acci_ref[...])


def _vq_argmin_body(x_ref, e_ref, x2_ref, idx_ref,
                    accv_ref, acci_ref, winv_ref, wini_ref):
    j = pl.program_id(1)
    e_blk = e_ref[...]
    dot = jax.lax.dot_general(
        x_ref[...], e_blk, (((1,), (1,)), ((), ())),
        preferred_element_type=jnp.float32)
    e2 = jnp.sum(e_blk * e_blk, axis=1)[None, :]
    d = (x2_ref[...] + e2) - 2.0 * dot

    @pl.when(j == 0)
    def _():
        accv_ref[...] = jnp.full_like(accv_ref[...], jnp.inf)
        acci_ref[...] = jnp.zeros_like(acci_ref[...])

    refs = (accv_ref, acci_ref, winv_ref, wini_ref)
    patterns = {
        0: [(0, 512, True, False)],
        1: [(0, 512, False, False)],
        2: [(0, 256, False, True), (256, 256, True, False)],
        3: [(0, 512, False, False)],
        4: [(0, 512, False, True)],
    }
    for pat, pieces in patterns.items():
        cond = j % 5 == pat
        if pat == 1:
            cond = jnp.logical_and(cond, j != NJ - 1)

        @pl.when(cond)
        def _(pieces=pieces):
            for (c0, w, st, cl) in pieces:
                _piece(d, j, c0, w, st, cl, *refs)

    @pl.when(j == NJ - 1)
    def _():
        _piece(d, j, 0, 512, False, True, *refs)
        idx_ref[...] = acci_ref[...]


@jax.jit
def kernel(x, embedding):
    x2 = jnp.sum(x ** 2, axis=1, keepdims=True)

    grid = (N_TOKENS // BM, NJ)
    idx2d = pl.pallas_call(
        _vq_argmin_body,
        grid=grid,
        in_specs=[
            pl.BlockSpec((BM, DIM), lambda i, j: (i, 0)),
            pl.BlockSpec((BN, DIM), lambda i, j: (j, 0)),
            pl.BlockSpec((BM, 1), lambda i, j: (i, 0)),
        ],
        out_specs=pl.BlockSpec((BM, 1), lambda i, j: (i, 0)),
        out_shape=jax.ShapeDtypeStruct((N_TOKENS, 1), jnp.int32),
        scratch_shapes=[
            pltpu.VMEM((BM, 1), jnp.float32),
            pltpu.VMEM((BM, 1), jnp.int32),
            pltpu.VMEM((BM, 1), jnp.float32),
            pltpu.VMEM((BM, 1), jnp.int32),
        ],
        compiler_params=pltpu.CompilerParams(
            dimension_semantics=("parallel", "arbitrary")),
    )(x, embedding, x2)

    indices = idx2d[:, 0]
    z_q = _sc_gather(embedding, indices)
    return (z_q, indices)


_B_PER_W = 128
_CHUNK = 32


def _sc_gather_body(table_hbm, idx_hbm, out_hbm, idx_v, rows_v, sem):
    wid = lax.axis_index("s") * 2 + lax.axis_index("c")
    base = wid * _B_PER_W
    pltpu.sync_copy(idx_hbm.at[pl.ds(base, _B_PER_W)], idx_v)
    for c in range(_B_PER_W // _CHUNK):
        pltpu.async_copy(
            table_hbm.at[idx_v.at[pl.ds(c * _CHUNK, _CHUNK)]],
            rows_v, sem).wait()
        pltpu.sync_copy(rows_v,
                        out_hbm.at[pl.ds(base + c * _CHUNK, _CHUNK)])


@functools.partial(jax.jit, static_argnames=())
def _sc_gather(embedding, indices):
    mesh = plsc.VectorSubcoreMesh(core_axis_name="c", subcore_axis_name="s")
    kern = functools.partial(
        pl.kernel,
        out_type=jax.ShapeDtypeStruct((N_TOKENS, DIM), jnp.float32),
        mesh=mesh,
        scratch_types=[
            pltpu.VMEM((_B_PER_W,), jnp.int32),
            pltpu.VMEM((_CHUNK, DIM), jnp.float32),
            pltpu.SemaphoreType.DMA,
        ],
    )(_sc_gather_body)
    return kern(embedding, indices)


# speedup vs baseline: 1.1264x; 1.0008x over previous
import functools

import jax
import jax.numpy as jnp
from jax import lax
from jax.experimental import pallas as pl
from jax.experimental.pallas import tpu as pltpu
from jax.experimental.pallas import tpu_sc as plsc

N_TOKENS = 4096
N_CODES = 16384
DIM = 2048
BM = 2048
BN = 512
NJ = N_CODES // BN


def _piece(d, j, c0, width, start, close, accv_ref, acci_ref,
           winv_ref, wini_ref):
    dp = d[:, c0:c0 + width]
    pmin = jnp.min(dp, axis=1, keepdims=True)
    cols = jax.lax.broadcasted_iota(jnp.int32, dp.shape, 1)
    ploc = jnp.min(jnp.where(dp == pmin, cols, N_CODES),
                   axis=1, keepdims=True)
    pidx = ploc + (j * BN + c0)
    if start:
        winv_ref[...] = pmin
        wini_ref[...] = pidx
    else:
        upd = pmin < winv_ref[...]
        winv_ref[...] = jnp.where(upd, pmin, winv_ref[...])
        wini_ref[...] = jnp.where(upd, pidx, wini_ref[...])
    if close:
        aupd = winv_ref[...] < accv_ref[...]
        accv_ref[...] = jnp.where(
            aupd, winv_ref[...].astype(jnp.bfloat16).astype(jnp.float32),
            accv_ref[...])
        acci_ref[...] = jnp.where(aupd, wini_ref[...], acci_ref[...])


def _vq_argmin_body(x_ref, e_ref, x2_ref, idx_ref,
                    accv_ref, acci_ref, winv_ref, wini_ref):
    j = pl.program_id(1)
    e_blk = e_ref[...]
    dot = jax.lax.dot_general(
        x_ref[...], e_blk, (((1,), (1,)), ((), ())),
        preferred_element_type=jnp.float32)
    e2 = jnp.sum(e_blk * e_blk, axis=1)[None, :]
    d = (x2_ref[...] + e2) - 2.0 * dot

    @pl.when(j == 0)
    def _():
        accv_ref[...] = jnp.full_like(accv_ref[...], jnp.inf)
        acci_ref[...] = jnp.zeros_like(acci_ref[...])

    refs = (accv_ref, acci_ref, winv_ref, wini_ref)
    patterns = {
        0: [(0, 512, True, False)],
        1: [(0, 512, False, False)],
        2: [(0, 256, False, True), (256, 256, True, False)],
        3: [(0, 512, False, False)],
        4: [(0, 512, False, True)],
    }
    for pat, pieces in patterns.items():
        cond = j % 5 == pat
        if pat == 1:
            cond = jnp.logical_and(cond, j != NJ - 1)

        @pl.when(cond)
        def _(pieces=pieces):
            for (c0, w, st, cl) in pieces:
                _piece(d, j, c0, w, st, cl, *refs)

    @pl.when(j == NJ - 1)
    def _():
        _piece(d, j, 0, 512, False, True, *refs)
        idx_ref[...] = acci_ref[...]


@jax.jit
def kernel(x, embedding):
    x2 = jnp.sum(x ** 2, axis=1, keepdims=True)

    grid = (N_TOKENS // BM, NJ)
    idx2d = pl.pallas_call(
        _vq_argmin_body,
        grid=grid,
        in_specs=[
            pl.BlockSpec((BM, DIM), lambda i, j: (i, 0)),
            pl.BlockSpec((BN, DIM), lambda i, j: (j, 0)),
            pl.BlockSpec((BM, 1), lambda i, j: (i, 0)),
        ],
        out_specs=pl.BlockSpec((BM, 1), lambda i, j: (i, 0)),
        out_shape=jax.ShapeDtypeStruct((N_TOKENS, 1), jnp.int32),
        scratch_shapes=[
            pltpu.VMEM((BM, 1), jnp.float32),
            pltpu.VMEM((BM, 1), jnp.int32),
            pltpu.VMEM((BM, 1), jnp.float32),
            pltpu.VMEM((BM, 1), jnp.int32),
        ],
        compiler_params=pltpu.CompilerParams(
            dimension_semantics=("parallel", "arbitrary")),
    )(x, embedding, x2)

    indices = idx2d[:, 0]
    z_q = _sc_gather(embedding, indices)
    return (z_q, indices)


_B_PER_W = 128
_CHUNK = 16


def _sc_gather_body(table_hbm, idx_hbm, out_hbm, idx_v, rows0, rows1,
                    sem0, sem1):
    wid = lax.axis_index("s") * 2 + lax.axis_index("c")
    base = wid * _B_PER_W
    pltpu.sync_copy(idx_hbm.at[pl.ds(base, _B_PER_W)], idx_v)
    rows = (rows0, rows1)
    sems = (sem0, sem1)
    nch = _B_PER_W // _CHUNK
    handles = [None, None]
    handles[0] = pltpu.async_copy(
        table_hbm.at[idx_v.at[pl.ds(0, _CHUNK)]], rows[0], sems[0])
    for c in range(nch):
        if c + 1 < nch:
            handles[(c + 1) % 2] = pltpu.async_copy(
                table_hbm.at[idx_v.at[pl.ds((c + 1) * _CHUNK, _CHUNK)]],
                rows[(c + 1) % 2], sems[(c + 1) % 2])
        handles[c % 2].wait()
        pltpu.sync_copy(rows[c % 2],
                        out_hbm.at[pl.ds(base + c * _CHUNK, _CHUNK)])


@functools.partial(jax.jit, static_argnames=())
def _sc_gather(embedding, indices):
    mesh = plsc.VectorSubcoreMesh(core_axis_name="c", subcore_axis_name="s")
    kern = functools.partial(
        pl.kernel,
        out_type=jax.ShapeDtypeStruct((N_TOKENS, DIM), jnp.float32),
        mesh=mesh,
        scratch_types=[
            pltpu.VMEM((_B_PER_W,), jnp.int32),
            pltpu.VMEM((_CHUNK, DIM), jnp.float32),
            pltpu.VMEM((_CHUNK, DIM), jnp.float32),
            pltpu.SemaphoreType.DMA,
            pltpu.SemaphoreType.DMA,
        ],
    )(_sc_gather_body)
    return kern(embedding, indices)
